# baseline (speedup 1.0000x reference)
import jax
import jax.numpy as jnp
from jax import lax
from jax.experimental import pallas as pl
from jax.experimental.pallas import tpu as pltpu
from jax.experimental.pallas import tpu_sc as plsc

B = 16384
EMB = 32
NC = 2
NS = 16
L = 16
NW = NC * NS
BPW = B // NW
CH = BPW // L


def _sc_body(users_hbm, items_hbm, utT_hbm, itT_hbm, bias_hbm, out_hbm,
             uidx_v, iidx_v, ubuf, ibuf, bbuf, out_v, sem):
    wid = lax.axis_index("s") * NC + lax.axis_index("c")
    base = wid * BPW

    pltpu.sync_copy(users_hbm.at[pl.ds(base, BPW)], uidx_v)
    pltpu.sync_copy(items_hbm.at[pl.ds(base, BPW)], iidx_v)

    cps = [pltpu.async_copy(bias_hbm.at[iidx_v], bbuf, sem)]
    for d in range(EMB):
        cps.append(pltpu.async_copy(utT_hbm.at[d].at[uidx_v], ubuf.at[d], sem))
        cps.append(pltpu.async_copy(itT_hbm.at[d].at[iidx_v], ibuf.at[d], sem))
    for cp in cps:
        cp.wait()

    def chunk(c, carry):
        sl = pl.ds(c * L, L)
        acc = bbuf[sl]
        for d in range(EMB):
            acc = acc + ubuf[d, sl] * ibuf[d, sl]
        out_v[sl] = acc
        return carry

    lax.fori_loop(0, CH, chunk, 0)
    pltpu.sync_copy(out_v, out_hbm.at[pl.ds(base, BPW)])


def kernel(users, items, user_table, item_table, bias_table):
    mesh = plsc.VectorSubcoreMesh(core_axis_name="c", subcore_axis_name="s")
    f = pl.kernel(
        _sc_body,
        out_type=jax.ShapeDtypeStruct((B,), jnp.float32),
        mesh=mesh,
        compiler_params=pltpu.CompilerParams(
            needs_layout_passes=False, use_tc_tiling_on_sc=False),
        scratch_types=[
            pltpu.VMEM((BPW,), jnp.int32),
            pltpu.VMEM((BPW,), jnp.int32),
            pltpu.VMEM((EMB, BPW), jnp.float32),
            pltpu.VMEM((EMB, BPW), jnp.float32),
            pltpu.VMEM((BPW,), jnp.float32),
            pltpu.VMEM((BPW,), jnp.float32),
            pltpu.SemaphoreType.DMA,
        ],
    )
    return f(users.astype(jnp.int32), items.astype(jnp.int32),
             user_table.T, item_table.T, jnp.reshape(bias_table, (-1,)))


# speedup vs baseline: 4.5023x; 4.5023x over previous
import jax
import jax.numpy as jnp
from jax import lax
from jax.experimental import pallas as pl
from jax.experimental.pallas import tpu as pltpu
from jax.experimental.pallas import tpu_sc as plsc

B = 16384
EMB = 32
NC = 2
NS = 16
L = 16
NW = NC * NS
BPW = B // NW
CH = BPW // L


def _sc_body(users_hbm, items_hbm, ut_hbm, it_hbm, bias_hbm, out_hbm,
             uidx_v, iidx_v, gu_v, gi_v, ubuf, ibuf, bbuf, out_v, sem):
    wid = lax.axis_index("s") * NC + lax.axis_index("c")
    base = wid * BPW

    pltpu.sync_copy(users_hbm.at[pl.ds(base, BPW)], uidx_v)
    pltpu.sync_copy(items_hbm.at[pl.ds(base, BPW)], iidx_v)

    bias_cp = pltpu.async_copy(bias_hbm.at[iidx_v], bbuf, sem)

    emb = jnp.full((L,), EMB, jnp.int32)

    def build(c, carry):
        sl = pl.ds(c * L, L)
        ub = uidx_v[sl] * emb
        ib = iidx_v[sl] * emb
        for d in range(EMB):
            gu_v[d, sl] = ub + d
            gi_v[d, sl] = ib + d
        return carry

    lax.fori_loop(0, CH, build, 0)

    cps = []
    for d in range(EMB):
        cps.append(pltpu.async_copy(ut_hbm.at[gu_v.at[d]], ubuf.at[d], sem))
        cps.append(pltpu.async_copy(it_hbm.at[gi_v.at[d]], ibuf.at[d], sem))
    for cp in cps:
        cp.wait()
    bias_cp.wait()

    def chunk(c, carry):
        sl = pl.ds(c * L, L)
        acc = bbuf[sl]
        for d in range(EMB):
            acc = acc + ubuf[d, sl] * ibuf[d, sl]
        out_v[sl] = acc
        return carry

    lax.fori_loop(0, CH, chunk, 0)
    pltpu.sync_copy(out_v, out_hbm.at[pl.ds(base, BPW)])


def kernel(users, items, user_table, item_table, bias_table):
    mesh = plsc.VectorSubcoreMesh(core_axis_name="c", subcore_axis_name="s")
    f = pl.kernel(
        _sc_body,
        out_type=jax.ShapeDtypeStruct((B,), jnp.float32),
        mesh=mesh,
        compiler_params=pltpu.CompilerParams(
            needs_layout_passes=False, use_tc_tiling_on_sc=False),
        scratch_types=[
            pltpu.VMEM((BPW,), jnp.int32),
            pltpu.VMEM((BPW,), jnp.int32),
            pltpu.VMEM((EMB, BPW), jnp.int32),
            pltpu.VMEM((EMB, BPW), jnp.int32),
            pltpu.VMEM((EMB, BPW), jnp.float32),
            pltpu.VMEM((EMB, BPW), jnp.float32),
            pltpu.VMEM((BPW,), jnp.float32),
            pltpu.VMEM((BPW,), jnp.float32),
            pltpu.SemaphoreType.DMA,
        ],
    )
    return f(users.astype(jnp.int32), items.astype(jnp.int32),
             jnp.reshape(user_table, (-1,)), jnp.reshape(item_table, (-1,)),
             jnp.reshape(bias_table, (-1,)))


# speedup vs baseline: 4.7175x; 1.0478x over previous
import jax
import jax.numpy as jnp
from jax import lax
from jax.experimental import pallas as pl
from jax.experimental.pallas import tpu as pltpu
from jax.experimental.pallas import tpu_sc as plsc

B = 16384
EMB = 32
NC = 2
NS = 16
L = 16
NW = NC * NS
BPW = B // NW
G = BPW // L
BW = 16


def _sc_body(users_hbm, items_hbm, ut_hbm, it_hbm, bt_hbm, out_hbm,
             uidx_v, iidx_v, gidx_v, urows_v, irows_v, brows_v, out_v, sem):
    wid = lax.axis_index("s") * NC + lax.axis_index("c")
    base = wid * BPW

    pltpu.sync_copy(users_hbm.at[pl.ds(base, BPW)], uidx_v)
    pltpu.sync_copy(items_hbm.at[pl.ds(base, BPW)], iidx_v)

    cp_u = pltpu.async_copy(ut_hbm.at[uidx_v], urows_v, sem)
    cp_i = pltpu.async_copy(it_hbm.at[iidx_v], irows_v, sem)

    def shift_chunk(g, carry):
        gidx_v[pl.ds(g * L, L)] = lax.shift_right_logical(
            iidx_v[pl.ds(g * L, L)], 4)
        return carry

    lax.fori_loop(0, G, shift_chunk, 0)

    cp_b = pltpu.async_copy(bt_hbm.at[gidx_v], brows_v, sem)
    cp_u.wait()
    cp_i.wait()
    cp_b.wait()

    lanes = lax.iota(jnp.int32, 16)
    low_mask = jnp.full((L,), BW - 1, jnp.int32)

    def group(g, carry):
        rows = g * L + lanes
        acc = jnp.zeros((L,), jnp.float32)
        for d in range(EMB):
            col = jnp.full((L,), d, jnp.int32)
            uv = plsc.load_gather(urows_v, [rows, col])
            iv = plsc.load_gather(irows_v, [rows, col])
            acc = acc + uv * iv
        bcol = iidx_v[pl.ds(g * L, L)] & low_mask
        bv = plsc.load_gather(brows_v, [rows, bcol])
        out_v[pl.ds(g * L, L)] = acc + bv
        return carry

    lax.fori_loop(0, G, group, 0)

    pltpu.sync_copy(out_v, out_hbm.at[pl.ds(base, BPW)])


def kernel(users, items, user_table, item_table, bias_table):
    n_items = bias_table.shape[0]
    mesh = plsc.VectorSubcoreMesh(core_axis_name="c", subcore_axis_name="s")
    f = pl.kernel(
        _sc_body,
        out_type=jax.ShapeDtypeStruct((B,), jnp.float32),
        mesh=mesh,
        compiler_params=pltpu.CompilerParams(
            needs_layout_passes=False, use_tc_tiling_on_sc=False),
        scratch_types=[
            pltpu.VMEM((BPW,), jnp.int32),
            pltpu.VMEM((BPW,), jnp.int32),
            pltpu.VMEM((BPW,), jnp.int32),
            pltpu.VMEM((BPW, EMB), jnp.float32),
            pltpu.VMEM((BPW, EMB), jnp.float32),
            pltpu.VMEM((BPW, BW), jnp.float32),
            pltpu.VMEM((BPW,), jnp.float32),
            pltpu.SemaphoreType.DMA,
        ],
    )
    bias2d = jnp.reshape(bias_table, (n_items // BW, BW))
    return f(users.astype(jnp.int32), items.astype(jnp.int32),
             user_table, item_table, bias2d)
